# baseline (speedup 1.0000x reference)
import functools

import jax
import jax.numpy as jnp
from jax import lax
from jax.experimental import pallas as pl
from jax.experimental.pallas import tpu as pltpu
from jax.experimental.pallas import tpu_sc as plsc

_NC = 2
_NS = 16
_NW = _NC * _NS
_L = 16

_SPLIT = 128
_WSEG = 16
_RBLK = 4096

_GDN = lax.GatherDimensionNumbers(
    offset_dims=(), collapsed_slice_dims=(0,), start_index_map=(0,))


def _perm(v, idx):
    return lax.gather(v, idx.reshape(_L, 1), _GDN, (1,),
                      mode=lax.GatherScatterMode.PROMISE_IN_BOUNDS)


def _lane_bcast(v, l):
    return _perm(v, jnp.full((_L,), l, jnp.int32))


def _allsum(v, liota):
    for c in (1, 2, 4, 8):
        v = v + _perm(v, liota ^ c)
    return v


def _sc_body(nrows, segs_per_w, af_hbm, bf_hbm, meta_hbm, wab_hbm, out_hbm,
             meta_v, w_v, rows_v, scores_v, out_v, sems):
    wid = lax.axis_index("s") * _NC + lax.axis_index("c")
    liota = lax.iota(jnp.int32, _L)
    ninf = jnp.full((_L,), -jnp.inf, jnp.float32)
    pltpu.sync_copy(wab_hbm, w_v)

    for br, feats in ((0, af_hbm), (1, bf_hbm)):
        pltpu.sync_copy(meta_hbm.at[br, wid], meta_v)
        mvec = meta_v[...]
        wch = [w_v[br, j] for j in range(8)]

        def seg_meta(k):
            base_vec = _lane_bcast(mvec, k)
            size_vec = _lane_bcast(
                mvec, jnp.minimum(segs_per_w + k, 2 * segs_per_w - 1))
            out_v[pl.ds(0, _L)] = size_vec.astype(jnp.float32)
            size_s = out_v[pl.ds(0, _L)][0].astype(jnp.int32)
            return base_vec, size_vec, size_s

        def dma_chunks(k, slot, do_start):
            kk = jnp.minimum(k, segs_per_w - 1)
            base_vec, _, size_s = seg_meta(kk)
            ng = jnp.where(k < segs_per_w, (size_s + (_L - 1)) // _L, 0)

            def one(i, _):
                idx = jnp.minimum(base_vec + i * _L + liota, nrows - 1)
                cp = pltpu.make_async_copy(
                    feats.at[idx], rows_v.at[slot, pl.ds(i * _L, _L)],
                    sems.at[slot])
                if do_start:
                    cp.start()
                else:
                    cp.wait()
                return 0

            lax.fori_loop(0, ng, one, 0)

        dma_chunks(0, 0, True)

        def seg_body(k, _):
            slot = lax.rem(k, 2)
            seg = jnp.where(lax.rem(k, 2) == 0, wid, _NW - 1 - wid) + _NW * k
            dma_chunks(k, slot, False)
            dma_chunks(k + 1, 1 - slot, True)
            _, size_vec, size_s = seg_meta(k)
            ng = (size_s + (_L - 1)) // _L

            def row_a(i, carry):
                m_vec, svec = carry
                for r in (2 * i, 2 * i + 1):
                    acc = rows_v[slot, r, pl.ds(0, _L)] * wch[0]
                    for j in range(1, 8):
                        acc = acc + rows_v[slot, r, pl.ds(j * _L, _L)] * wch[j]
                    scv = _allsum(acc, liota)
                    scv = jnp.where(scv >= 0.0, scv, 0.2 * scv)
                    m_vec = jnp.maximum(
                        m_vec, jnp.where(r < size_s, scv, ninf))
                    svec = jnp.where(liota == lax.rem(r, _L), scv, svec)

                @pl.when(lax.rem(i, _L // 2) == _L // 2 - 1)
                def _():
                    scores_v[pl.ds((2 * i // _L) * _L, _L)] = svec

                return m_vec, svec

            m_vec, _ = lax.fori_loop(
                0, ng * (_L // 2), row_a,
                (ninf, jnp.zeros((_L,), jnp.float32)))
            m_vec = jnp.where(m_vec > -jnp.inf, m_vec, 0.0)

            def e_pass(g, s_acc):
                sv = scores_v[pl.ds(g * _L, _L)]
                lvalid = (g * _L + liota) < size_vec
                ev = jnp.where(lvalid, jnp.exp(sv - m_vec), 0.0)
                scores_v[pl.ds(g * _L, _L)] = ev
                return s_acc + ev

            s_vec = lax.fori_loop(0, ng, e_pass, jnp.zeros((_L,), jnp.float32))
            s_all = _allsum(s_vec, liota)
            inv_vec = 1.0 / jnp.where(s_all > 0.0, s_all, 1.0)

            def row_b(g, accs):
                av = scores_v[pl.ds(g * _L, _L)] * inv_vec
                for l in range(_L):
                    ab = _lane_bcast(av, l)
                    r = g * _L + l
                    accs = tuple(
                        accs[j] + rows_v[slot, r, pl.ds(j * _L, _L)] * ab
                        for j in range(8))
                return accs

            accs = lax.fori_loop(
                0, ng, row_b,
                tuple(jnp.zeros((_L,), jnp.float32) for _ in range(8)))
            for j in range(8):
                out_v[pl.ds(j * _L, _L)] = accs[j]
            pltpu.sync_copy(out_v, out_hbm.at[br, seg])
            return 0

        lax.fori_loop(0, segs_per_w, seg_body, 0)


def _tc_body(scal, af_hbm, bf_hbm, lo_a, hi_a, lo_b, hi_b, gf_blk, wa, wb,
             out_blk, va, vb, sems):
    w = pl.program_id(0)
    nwin = pl.num_programs(0)
    slot = lax.rem(w, 2)

    def issue(win, slot):
        pltpu.make_async_copy(
            af_hbm.at[pl.ds(scal[0, win], _RBLK)], va.at[slot], sems.at[0, slot]
        ).start()
        pltpu.make_async_copy(
            bf_hbm.at[pl.ds(scal[1, win], _RBLK)], vb.at[slot], sems.at[1, slot]
        ).start()

    @pl.when(w == 0)
    def _():
        issue(0, 0)

    @pl.when(w + 1 < nwin)
    def _():
        issue(w + 1, 1 - slot)

    pltpu.make_async_copy(
        af_hbm.at[pl.ds(scal[0, w], _RBLK)], va.at[slot], sems.at[0, slot]
    ).wait()
    pltpu.make_async_copy(
        bf_hbm.at[pl.ds(scal[1, w], _RBLK)], vb.at[slot], sems.at[1, slot]
    ).wait()

    riota = lax.broadcasted_iota(jnp.int32, (_RBLK, _WSEG), 0)

    def branch(x, lo, hi, w_rep):
        score = lax.dot_general(
            x, w_rep[...], (((1,), (0,)), ((), ())),
            preferred_element_type=jnp.float32)
        score = jnp.where(score >= 0.0, score, 0.2 * score)
        mask = (riota >= lo[0]) & (riota < hi[0])
        m = jnp.max(jnp.where(mask, score, -jnp.inf), axis=0, keepdims=True)
        m = jnp.where(jnp.isfinite(m), m, 0.0)
        e = jnp.where(mask, jnp.exp(score - m), 0.0)
        s = jnp.sum(e, axis=0, keepdims=True)
        alpha = e / jnp.where(s > 0.0, s, 1.0)
        return lax.dot_general(
            alpha, x, (((0,), (0,)), ((), ())),
            preferred_element_type=jnp.float32)

    ra = branch(va[slot], lo_a, hi_a, wa)
    rb = branch(vb[slot], lo_b, hi_b, wb)
    out_blk[0, :, 0:128] = ra
    out_blk[0, :, 128:256] = rb
    out_blk[0, :, 256:384] = gf_blk[0]


def kernel(atom_feats, atom_sizes, bond_feats, bond_sizes, global_feats, w_atom, w_bond):
    N, D = atom_feats.shape
    B = global_feats.shape[0]
    segs_per_w = _SPLIT // _NW
    nwin = (B - _SPLIT) // _WSEG

    def mk_starts(sizes):
        sizes = sizes.astype(jnp.int32)
        cs = jnp.cumsum(sizes)
        return sizes, jnp.concatenate([jnp.zeros((1,), jnp.int32), cs[:-1]])

    sz_a, st_a = mk_starts(atom_sizes)
    sz_b, st_b = mk_starts(bond_sizes)

    w_col = jnp.arange(_NW)[:, None]
    k_row = jnp.arange(segs_per_w)[None, :]
    idxs = jnp.where(k_row % 2 == 0, w_col, _NW - 1 - w_col) + _NW * k_row

    def mk_meta(sizes, starts):
        pad = jnp.zeros((_NW, _L - 2 * segs_per_w), jnp.int32)
        return jnp.concatenate([starts[idxs], sizes[idxs], pad], axis=1)

    meta = jnp.stack([mk_meta(sz_a, st_a), mk_meta(sz_b, st_b)])
    wab = jnp.stack([w_atom.reshape(8, _L), w_bond.reshape(8, _L)])

    mesh = plsc.VectorSubcoreMesh(core_axis_name="c", subcore_axis_name="s")
    sc_run = functools.partial(
        pl.kernel,
        mesh=mesh,
        out_type=jax.ShapeDtypeStruct((2, _SPLIT, D), jnp.float32),
        scratch_types=[
            pltpu.VMEM((_L,), jnp.int32),
            pltpu.VMEM((2, 8, _L), jnp.float32),
            pltpu.VMEM((2, 256, D), jnp.float32),
            pltpu.VMEM((256,), jnp.float32),
            pltpu.VMEM((D,), jnp.float32),
            pltpu.SemaphoreType.DMA((2,)),
        ],
    )(functools.partial(_sc_body, N, segs_per_w))

    def mk_tc(sizes, starts):
        base_c = jnp.minimum(starts[_SPLIT:: _WSEG], N - _RBLK)
        lo = starts[_SPLIT:].reshape(nwin, _WSEG) - base_c[:, None]
        hi = lo + sizes[_SPLIT:].reshape(nwin, _WSEG)
        return base_c, lo.reshape(nwin, 1, _WSEG), hi.reshape(nwin, 1, _WSEG)

    ba, lo_a, hi_a = mk_tc(sz_a, st_a)
    bb, lo_b, hi_b = mk_tc(sz_b, st_b)
    scal = jnp.stack([ba, bb])

    win_spec = pl.BlockSpec((1, 1, _WSEG), lambda w, s: (w, 0, 0))
    grid_spec = pltpu.PrefetchScalarGridSpec(
        num_scalar_prefetch=1,
        grid=(nwin,),
        in_specs=[
            pl.BlockSpec(memory_space=pltpu.MemorySpace.HBM),
            pl.BlockSpec(memory_space=pltpu.MemorySpace.HBM),
            win_spec, win_spec,
            win_spec, win_spec,
            pl.BlockSpec((1, _WSEG, D), lambda w, s: (w, 0, 0)),
            pl.BlockSpec((D, _WSEG), lambda w, s: (0, 0)),
            pl.BlockSpec((D, _WSEG), lambda w, s: (0, 0)),
        ],
        out_specs=pl.BlockSpec((1, _WSEG, 3 * D), lambda w, s: (w, 0, 0)),
        scratch_shapes=[
            pltpu.VMEM((2, _RBLK, D), jnp.float32),
            pltpu.VMEM((2, _RBLK, D), jnp.float32),
            pltpu.SemaphoreType.DMA((2, 2)),
        ],
    )
    tc_out = pl.pallas_call(
        _tc_body,
        grid_spec=grid_spec,
        out_shape=jax.ShapeDtypeStruct((nwin, _WSEG, 3 * D), jnp.float32),
    )(scal, atom_feats, bond_feats, lo_a, hi_a, lo_b, hi_b,
      global_feats[_SPLIT:].reshape(nwin, _WSEG, D),
      jnp.broadcast_to(w_atom.reshape(D, 1), (D, _WSEG)),
      jnp.broadcast_to(w_bond.reshape(D, 1), (D, _WSEG)))

    pooled = sc_run(atom_feats, bond_feats, meta, wab)

    top = jnp.concatenate(
        [pooled[0], pooled[1], global_feats[:_SPLIT]], axis=1)
    return jnp.concatenate([top, tc_out.reshape(B - _SPLIT, 3 * D)], axis=0)


# speedup vs baseline: 1.0744x; 1.0744x over previous
import functools

import jax
import jax.numpy as jnp
from jax import lax
from jax.experimental import pallas as pl
from jax.experimental.pallas import tpu as pltpu
from jax.experimental.pallas import tpu_sc as plsc

_NC = 2
_NS = 16
_NW = _NC * _NS
_L = 16

_SPLIT = 128
_WSEG = 16
_RBLK = 4096

_GDN = lax.GatherDimensionNumbers(
    offset_dims=(), collapsed_slice_dims=(0,), start_index_map=(0,))


def _perm(v, idx):
    return lax.gather(v, idx.reshape(_L, 1), _GDN, (1,),
                      mode=lax.GatherScatterMode.PROMISE_IN_BOUNDS)


def _lane_bcast(v, l):
    return _perm(v, jnp.full((_L,), l, jnp.int32))


def _allsum(v, liota):
    for c in (1, 2, 4, 8):
        v = v + _perm(v, liota ^ c)
    return v


def _sc_body(nrows, segs_per_w, af_hbm, bf_hbm, meta_hbm, wab_hbm, out_hbm,
             meta_v, w_v, rows_v, scores_v, out_v, sems):
    wid = lax.axis_index("s") * _NC + lax.axis_index("c")
    liota = lax.iota(jnp.int32, _L)
    ninf = jnp.full((_L,), -jnp.inf, jnp.float32)
    pltpu.sync_copy(wab_hbm, w_v)

    for br, feats in ((0, af_hbm), (1, bf_hbm)):
        pltpu.sync_copy(meta_hbm.at[br, wid], meta_v)
        mvec = meta_v[...]
        wch = [w_v[br, j] for j in range(8)]

        def seg_meta(k):
            base_vec = _lane_bcast(mvec, k)
            size_vec = _lane_bcast(
                mvec, jnp.minimum(segs_per_w + k, 2 * segs_per_w - 1))
            out_v[pl.ds(0, _L)] = size_vec.astype(jnp.float32)
            size_s = out_v[pl.ds(0, _L)][0].astype(jnp.int32)
            return base_vec, size_vec, size_s

        def dma_chunks(k, slot, do_start):
            kk = jnp.minimum(k, segs_per_w - 1)
            base_vec, _, size_s = seg_meta(kk)
            ng = jnp.where(k < segs_per_w, (size_s + (_L - 1)) // _L, 0)

            def one(i, _):
                idx = jnp.minimum(base_vec + i * _L + liota, nrows - 1)
                cp = pltpu.make_async_copy(
                    feats.at[idx], rows_v.at[slot, pl.ds(i * _L, _L)],
                    sems.at[slot])
                if do_start:
                    cp.start()
                else:
                    cp.wait()
                return 0

            lax.fori_loop(0, ng, one, 0)

        dma_chunks(0, 0, True)

        def seg_body(k, _):
            slot = lax.rem(k, 2)
            seg = jnp.where(lax.rem(k, 2) == 0, wid, _NW - 1 - wid) + _NW * k
            dma_chunks(k, slot, False)
            dma_chunks(k + 1, 1 - slot, True)
            _, size_vec, size_s = seg_meta(k)
            ng = (size_s + (_L - 1)) // _L

            def row_a(i, carry):
                m_vec, svec = carry
                for r in (2 * i, 2 * i + 1):
                    acc = rows_v[slot, r, pl.ds(0, _L)] * wch[0]
                    for j in range(1, 8):
                        acc = acc + rows_v[slot, r, pl.ds(j * _L, _L)] * wch[j]
                    scv = _allsum(acc, liota)
                    scv = jnp.where(scv >= 0.0, scv, 0.2 * scv)
                    m_vec = jnp.maximum(
                        m_vec, jnp.where(r < size_s, scv, ninf))
                    svec = jnp.where(liota == lax.rem(r, _L), scv, svec)

                @pl.when(lax.rem(i, _L // 2) == _L // 2 - 1)
                def _():
                    scores_v[pl.ds((2 * i // _L) * _L, _L)] = svec

                return m_vec, svec

            m_vec, _ = lax.fori_loop(
                0, ng * (_L // 2), row_a,
                (ninf, jnp.zeros((_L,), jnp.float32)))
            m_vec = jnp.where(m_vec > -jnp.inf, m_vec, 0.0)

            def e_pass(g, s_acc):
                sv = scores_v[pl.ds(g * _L, _L)]
                lvalid = (g * _L + liota) < size_vec
                ev = jnp.where(lvalid, jnp.exp(sv - m_vec), 0.0)
                scores_v[pl.ds(g * _L, _L)] = ev
                return s_acc + ev

            s_vec = lax.fori_loop(0, ng, e_pass, jnp.zeros((_L,), jnp.float32))
            s_all = _allsum(s_vec, liota)
            inv_vec = 1.0 / jnp.where(s_all > 0.0, s_all, 1.0)

            def row_b(g, accs):
                av = scores_v[pl.ds(g * _L, _L)] * inv_vec
                for l in range(_L):
                    ab = _lane_bcast(av, l)
                    r = g * _L + l
                    accs = tuple(
                        accs[j] + rows_v[slot, r, pl.ds(j * _L, _L)] * ab
                        for j in range(8))
                return accs

            accs = lax.fori_loop(
                0, ng, row_b,
                tuple(jnp.zeros((_L,), jnp.float32) for _ in range(8)))
            for j in range(8):
                out_v[pl.ds(j * _L, _L)] = accs[j]
            pltpu.sync_copy(out_v, out_hbm.at[br, seg])
            return 0

        lax.fori_loop(0, segs_per_w, seg_body, 0)


def _tc_body(scal, af_hbm, bf_hbm, lo_a, hi_a, lo_b, hi_b, gf_blk, wa, wb,
             out_blk, va, vb, sems):
    w = pl.program_id(0)
    nwin = pl.num_programs(0)
    slot = lax.rem(w, 2)

    def issue(win, slot):
        pltpu.make_async_copy(
            af_hbm.at[pl.ds(scal[0, win], _RBLK)], va.at[slot], sems.at[0, slot]
        ).start()
        pltpu.make_async_copy(
            bf_hbm.at[pl.ds(scal[1, win], _RBLK)], vb.at[slot], sems.at[1, slot]
        ).start()

    @pl.when(w == 0)
    def _():
        issue(0, 0)

    @pl.when(w + 1 < nwin)
    def _():
        issue(w + 1, 1 - slot)

    pltpu.make_async_copy(
        af_hbm.at[pl.ds(scal[0, w], _RBLK)], va.at[slot], sems.at[0, slot]
    ).wait()
    pltpu.make_async_copy(
        bf_hbm.at[pl.ds(scal[1, w], _RBLK)], vb.at[slot], sems.at[1, slot]
    ).wait()

    riota = lax.broadcasted_iota(jnp.int32, (_RBLK, _WSEG), 0)

    def branch(x, lo, hi, w_row):
        score = jnp.sum(x * w_row, axis=1, keepdims=True)
        score = jnp.where(score >= 0.0, score, 0.2 * score)
        mask = (riota >= lo[0]) & (riota < hi[0])
        m = jnp.max(jnp.where(mask, score, -jnp.inf), axis=0, keepdims=True)
        m = jnp.where(jnp.isfinite(m), m, 0.0)
        e = jnp.where(mask, jnp.exp(score - m), 0.0)
        s = jnp.sum(e, axis=0, keepdims=True)
        alpha = e / jnp.where(s > 0.0, s, 1.0)
        return lax.dot_general(
            alpha, x, (((0,), (0,)), ((), ())),
            preferred_element_type=jnp.float32)

    ra = branch(va[slot], lo_a, hi_a, wa[...])
    rb = branch(vb[slot], lo_b, hi_b, wb[...])
    out_blk[0, :, 0:128] = ra
    out_blk[0, :, 128:256] = rb
    out_blk[0, :, 256:384] = gf_blk[0]


def kernel(atom_feats, atom_sizes, bond_feats, bond_sizes, global_feats, w_atom, w_bond):
    N, D = atom_feats.shape
    B = global_feats.shape[0]
    segs_per_w = _SPLIT // _NW
    nwin = (B - _SPLIT) // _WSEG

    def mk_starts(sizes):
        sizes = sizes.astype(jnp.int32)
        cs = jnp.cumsum(sizes)
        return sizes, jnp.concatenate([jnp.zeros((1,), jnp.int32), cs[:-1]])

    sz_a, st_a = mk_starts(atom_sizes)
    sz_b, st_b = mk_starts(bond_sizes)

    w_col = jnp.arange(_NW)[:, None]
    k_row = jnp.arange(segs_per_w)[None, :]
    idxs = jnp.where(k_row % 2 == 0, w_col, _NW - 1 - w_col) + _NW * k_row

    def mk_meta(sizes, starts):
        pad = jnp.zeros((_NW, _L - 2 * segs_per_w), jnp.int32)
        return jnp.concatenate([starts[idxs], sizes[idxs], pad], axis=1)

    meta = jnp.stack([mk_meta(sz_a, st_a), mk_meta(sz_b, st_b)])
    wab = jnp.stack([w_atom.reshape(8, _L), w_bond.reshape(8, _L)])

    mesh = plsc.VectorSubcoreMesh(core_axis_name="c", subcore_axis_name="s")
    sc_run = functools.partial(
        pl.kernel,
        mesh=mesh,
        out_type=jax.ShapeDtypeStruct((2, _SPLIT, D), jnp.float32),
        scratch_types=[
            pltpu.VMEM((_L,), jnp.int32),
            pltpu.VMEM((2, 8, _L), jnp.float32),
            pltpu.VMEM((2, 256, D), jnp.float32),
            pltpu.VMEM((256,), jnp.float32),
            pltpu.VMEM((D,), jnp.float32),
            pltpu.SemaphoreType.DMA((2,)),
        ],
    )(functools.partial(_sc_body, N, segs_per_w))

    def mk_tc(sizes, starts):
        base_c = jnp.minimum(starts[_SPLIT:: _WSEG], N - _RBLK)
        lo = starts[_SPLIT:].reshape(nwin, _WSEG) - base_c[:, None]
        hi = lo + sizes[_SPLIT:].reshape(nwin, _WSEG)
        return base_c, lo.reshape(nwin, 1, _WSEG), hi.reshape(nwin, 1, _WSEG)

    ba, lo_a, hi_a = mk_tc(sz_a, st_a)
    bb, lo_b, hi_b = mk_tc(sz_b, st_b)
    scal = jnp.stack([ba, bb])

    win_spec = pl.BlockSpec((1, 1, _WSEG), lambda w, s: (w, 0, 0))
    grid_spec = pltpu.PrefetchScalarGridSpec(
        num_scalar_prefetch=1,
        grid=(nwin,),
        in_specs=[
            pl.BlockSpec(memory_space=pltpu.MemorySpace.HBM),
            pl.BlockSpec(memory_space=pltpu.MemorySpace.HBM),
            win_spec, win_spec,
            win_spec, win_spec,
            pl.BlockSpec((1, _WSEG, D), lambda w, s: (w, 0, 0)),
            pl.BlockSpec((1, D), lambda w, s: (0, 0)),
            pl.BlockSpec((1, D), lambda w, s: (0, 0)),
        ],
        out_specs=pl.BlockSpec((1, _WSEG, 3 * D), lambda w, s: (w, 0, 0)),
        scratch_shapes=[
            pltpu.VMEM((2, _RBLK, D), jnp.float32),
            pltpu.VMEM((2, _RBLK, D), jnp.float32),
            pltpu.SemaphoreType.DMA((2, 2)),
        ],
    )
    tc_out = pl.pallas_call(
        _tc_body,
        grid_spec=grid_spec,
        out_shape=jax.ShapeDtypeStruct((nwin, _WSEG, 3 * D), jnp.float32),
    )(scal, atom_feats, bond_feats, lo_a, hi_a, lo_b, hi_b,
      global_feats[_SPLIT:].reshape(nwin, _WSEG, D),
      w_atom.reshape(1, D), w_bond.reshape(1, D))

    pooled = sc_run(atom_feats, bond_feats, meta, wab)

    top = jnp.concatenate(
        [pooled[0], pooled[1], global_feats[:_SPLIT]], axis=1)
    return jnp.concatenate([top, tc_out.reshape(B - _SPLIT, 3 * D)], axis=0)
